# baseline (speedup 1.0000x reference)
import functools

import jax
import jax.numpy as jnp
from jax import lax
from jax.experimental import pallas as pl
from jax.experimental.pallas import tpu as pltpu
from jax.experimental.pallas import tpu_sc as plsc

N = 10000
E = 320000
D = 128
H = 128
C = 2
K_L = 5
K_G = 10

NTILES = 16
NP = 10240
NPT = NP // NTILES
EPT = E // NTILES
TREP = 16 * (K_G + 1)
EDGE_UNROLL = 50


def _rsqrt_newton(d):
    i = plsc.bitcast(d, jnp.int32)
    i = jnp.int32(0x5F3759DF) - lax.shift_right_logical(i, 1)
    y = plsc.bitcast(i, jnp.float32)
    for _ in range(4):
        y = y * (1.5 - 0.5 * d * y * y)
    return y


def _tc_mlp_body(x_ref, w1l_ref, b1l_ref, w2l_ref, b2l_ref,
                 w1g_ref, b1g_ref, w2g_ref, b2g_ref,
                 h1l_ref, h1g_ref, z4_ref):
    x = x_ref[...]
    for b, (w1_ref, b1_ref, w2_ref, b2_ref, h1_ref) in enumerate((
        (w1l_ref, b1l_ref, w2l_ref, b2l_ref, h1l_ref),
        (w1g_ref, b1g_ref, w2g_ref, b2g_ref, h1g_ref),
    )):
        h1 = lax.dot_general(x, w1_ref[...], (((1,), (1,)), ((), ())))
        h1 = jnp.maximum(h1 + b1_ref[...][None, :], 0.0)
        h1_ref[...] = h1
        z = lax.dot_general(w2_ref[...], h1, (((1,), (1,)), ((), ())))
        z = jnp.pad(z + b2_ref[...][:, None], ((0, 0), (0, NP - N)))
        z4_ref[2 * b:2 * b + 2, :] = z


_tc_mlp = pl.pallas_call(
    _tc_mlp_body,
    out_shape=(
        jax.ShapeDtypeStruct((N, H), jnp.float32),
        jax.ShapeDtypeStruct((N, H), jnp.float32),
        jax.ShapeDtypeStruct((4, NP), jnp.float32),
    ),
)


def _tc_lsm_body(h2_ref, pl_ref, pg_ref):
    v = h2_ref[...]
    for b, out_ref in ((0, pl_ref), (1, pg_ref)):
        vb = v[2 * b:2 * b + 2, :]
        m = jnp.max(vb, axis=0, keepdims=True)
        lse = m + jnp.log(jnp.sum(jnp.exp(vb - m), axis=0, keepdims=True))
        out_ref[...] = vb - lse


_tc_lsm = pl.pallas_call(
    _tc_lsm_body,
    out_shape=(
        jax.ShapeDtypeStruct((C, NP), jnp.float32),
        jax.ShapeDtypeStruct((C, NP), jnp.float32),
    ),
)


_sc_mesh = plsc.VectorSubcoreMesh(core_axis_name="c", subcore_axis_name="s")


@functools.partial(
    pl.kernel,
    out_type=jax.ShapeDtypeStruct((4 * NP,), jnp.float32),
    mesh=_sc_mesh,
    compiler_params=pltpu.CompilerParams(needs_layout_passes=False),
    scratch_types=[
        pltpu.VMEM((EPT,), jnp.int32),
        pltpu.VMEM((EPT,), jnp.int32),
        pltpu.VMEM((NP,), jnp.float32),
        pltpu.VMEM((NP,), jnp.float32),
        pltpu.VMEM((NP,), jnp.float32),
        pltpu.VMEM((NP,), jnp.float32),
        pltpu.VMEM((NTILES, NPT), jnp.float32),
        pltpu.VMEM((NTILES, NPT), jnp.float32),
        pltpu.VMEM((NPT,), jnp.float32),
        pltpu.VMEM((NPT,), jnp.float32),
        pltpu.VMEM((NPT,), jnp.float32),
        pltpu.VMEM((NPT,), jnp.float32),
        pltpu.VMEM((NPT,), jnp.float32),
        pltpu.VMEM((NPT,), jnp.float32),
        pltpu.VMEM((TREP,), jnp.float32),
        pltpu.SemaphoreType.DMA,
        pltpu.SemaphoreType.DMA,
        pltpu.SemaphoreType.DMA,
        pltpu.SemaphoreType.DMA,
        pltpu.VMEM_SHARED((2, NTILES, NP), jnp.float32),
        pltpu.VMEM_SHARED((2, NP), jnp.float32),
    ],
)
def _sc_prop(src_hbm, dst_hbm, z_all, temp_rep, out,
             src_v, dst_v, y0_v, y1_v, s0_v, s1_v, part_v, part2_v,
             ysl0_v, ysl1_v, hid0_v, hid1_v, dinv_v, red_v, tsp_v,
             sem0, sem1, sem2, sem3, sh_p, sh_y):
    c = lax.axis_index("c")
    sid = lax.axis_index("s")
    zeros16 = jnp.zeros((16,), jnp.float32)
    ones16 = jnp.ones((16,), jnp.float32)
    n0 = sid * NPT

    pltpu.sync_copy(src_hbm.at[pl.ds(sid * EPT, EPT)], src_v)
    pltpu.sync_copy(dst_hbm.at[pl.ds(sid * EPT, EPT)], dst_v)
    pltpu.sync_copy(temp_rep.at[pl.ds(c * TREP, TREP)], tsp_v)

    def zero_buf(buf):
        @plsc.parallel_loop(0, NP // 16, 1, unroll=8)
        def zero_body(i):
            buf[pl.ds(i * 16, 16)] = zeros16

    def zero_s(_=None):
        zero_buf(s0_v)
        zero_buf(s1_v)

    def sum_slab(slab, dst_buf):
        @plsc.parallel_loop(0, NPT // 16, 1, unroll=4)
        def red_body(i):
            acc = slab[0, pl.ds(i * 16, 16)]
            for t in range(1, NTILES):
                acc = acc + slab[t, pl.ds(i * 16, 16)]
            dst_buf[pl.ds(i * 16, 16)] = acc

    def reduce_partials(ch, dst_buf):
        pltpu.sync_copy(sh_p.at[ch, pl.ds(0, NTILES), pl.ds(n0, NPT)],
                        part_v)
        sum_slab(part_v, dst_buf)

    zero_s()

    @plsc.parallel_loop(0, EPT // 16, 1, unroll=EDGE_UNROLL)
    def deg_body(i):
        d16 = dst_v[pl.ds(i * 16, 16)]
        plsc.addupdate_scatter(s0_v, [d16], ones16)
    pltpu.sync_copy(s0_v, sh_p.at[0, sid])
    plsc.subcore_barrier()

    reduce_partials(0, red_v)

    def dinv_body(i, _):
        d = red_v[pl.ds(i * 16, 16)] + 1.0
        dinv_v[pl.ds(i * 16, 16)] = _rsqrt_newton(d)
        return 0

    lax.fori_loop(0, NPT // 16, dinv_body, 0)

    @plsc.parallel_loop(0, NP // 16, 1, unroll=8)
    def zero_s0_body(i):
        s0_v[pl.ds(i * 16, 16)] = zeros16

    coef0 = tsp_v[pl.ds(0, 16)]
    for ch, (ysl, hid) in enumerate(((ysl0_v, hid0_v), (ysl1_v, hid1_v))):
        pltpu.sync_copy(z_all.at[pl.ds((2 * c + ch) * NP + n0, NPT)], ysl)

        def init_body(i, _, ysl=ysl, hid=hid):
            x0 = ysl[pl.ds(i * 16, 16)]
            hid[pl.ds(i * 16, 16)] = coef0 * x0
            ysl[pl.ds(i * 16, 16)] = dinv_v[pl.ds(i * 16, 16)] * x0
            return 0

        lax.fori_loop(0, NPT // 16, init_body, 0)
        pltpu.sync_copy(ysl, sh_y.at[ch, pl.ds(n0, NPT)])
    plsc.subcore_barrier()
    pltpu.sync_copy(sh_y.at[0], y0_v)
    pltpu.sync_copy(sh_y.at[1], y1_v)

    def step_body(k, _):
        @plsc.parallel_loop(0, EPT // 16, 1, unroll=EDGE_UNROLL)
        def edge_body(i):
            s16 = src_v[pl.ds(i * 16, 16)]
            d16 = dst_v[pl.ds(i * 16, 16)]
            g0 = plsc.load_gather(y0_v, [s16])
            g1 = plsc.load_gather(y1_v, [s16])
            plsc.addupdate_scatter(s0_v, [d16], g0)
            plsc.addupdate_scatter(s1_v, [d16], g1)

        cp0 = pltpu.async_copy(s0_v, sh_p.at[0, sid], sem0)
        cp1 = pltpu.async_copy(s1_v, sh_p.at[1, sid], sem1)
        cp0.wait()
        zero_buf(s0_v)
        cp1.wait()
        zero_buf(s1_v)
        plsc.subcore_barrier()

        coef = tsp_v[pl.ds(16 * (k + 1), 16)]
        sl0 = pltpu.async_copy(sh_p.at[0, pl.ds(0, NTILES), pl.ds(n0, NPT)],
                               part_v, sem0)
        sl1 = pltpu.async_copy(sh_p.at[1, pl.ds(0, NTILES), pl.ds(n0, NPT)],
                               part2_v, sem1)
        pubs = []
        for ch, (ysl, hid, slab, sl) in enumerate((
                (ysl0_v, hid0_v, part_v, sl0),
                (ysl1_v, hid1_v, part2_v, sl1))):
            sl.wait()
            sum_slab(slab, red_v)

            @plsc.parallel_loop(0, NPT // 16, 1, unroll=4)
            def upd_body(i, ysl=ysl, hid=hid, coef=coef):
                dv = dinv_v[pl.ds(i * 16, 16)]
                t = red_v[pl.ds(i * 16, 16)] + ysl[pl.ds(i * 16, 16)]
                xp = dv * t
                hid[pl.ds(i * 16, 16)] = hid[pl.ds(i * 16, 16)] + coef * xp
                ysl[pl.ds(i * 16, 16)] = dv * xp

            pubs.append(pltpu.async_copy(
                ysl, sh_y.at[ch, pl.ds(n0, NPT)], sem2 if ch == 0 else sem3))

        for p in pubs:
            p.wait()
        plsc.subcore_barrier()
        rd0 = pltpu.async_copy(sh_y.at[0], y0_v, sem0)
        rd1 = pltpu.async_copy(sh_y.at[1], y1_v, sem1)
        rd0.wait()
        rd1.wait()
        return 0

    lax.fori_loop(0, K_G, step_body, 0)

    pltpu.sync_copy(hid0_v, out.at[pl.ds(2 * c * NP + n0, NPT)])
    pltpu.sync_copy(hid1_v, out.at[pl.ds((2 * c + 1) * NP + n0, NPT)])


def kernel(x, edge_index, W1_l, b1_l, W2_l, b2_l, temp_l,
           W1_g, b1_g, W2_g, b2_g, temp_g):
    h1_l, h1_g, z4 = _tc_mlp(x, W1_l, b1_l, W2_l, b2_l,
                             W1_g, b1_g, W2_g, b2_g)

    z_all = z4.reshape(-1)
    tl = jnp.pad(temp_l, (0, K_G + 1 - temp_l.shape[0]))
    tg = jnp.pad(temp_g, (0, K_G + 1 - temp_g.shape[0]))
    temp_rep = jnp.concatenate([
        jnp.repeat(tl, 16), jnp.repeat(tg, 16)])

    hid = _sc_prop(edge_index[0], edge_index[1], z_all, temp_rep)
    hid2d = hid.reshape(4, NP)

    plT, pgT = _tc_lsm(hid2d)

    h2_l = hid2d[0:2, :N].T
    h2_g = hid2d[2:4, :N].T
    p_l = plT[:, :N].T
    p_g = pgT[:, :N].T
    return (h1_l, h2_l, p_l, h1_g, h2_g, p_g)


# speedup vs baseline: 1.0520x; 1.0520x over previous
import functools

import jax
import jax.numpy as jnp
from jax import lax
from jax.experimental import pallas as pl
from jax.experimental.pallas import tpu as pltpu
from jax.experimental.pallas import tpu_sc as plsc

N = 10000
E = 320000
D = 128
H = 128
C = 2
K_L = 5
K_G = 10

NTILES = 16
NP = 10240
NPT = NP // NTILES
EPT = E // NTILES
TREP = 16 * (K_G + 1)
EDGE_UNROLL = 25


def _rsqrt_newton(d):
    i = plsc.bitcast(d, jnp.int32)
    i = jnp.int32(0x5F3759DF) - lax.shift_right_logical(i, 1)
    y = plsc.bitcast(i, jnp.float32)
    for _ in range(4):
        y = y * (1.5 - 0.5 * d * y * y)
    return y


def _tc_mlp_body(x_ref, w1l_ref, b1l_ref, w2l_ref, b2l_ref,
                 w1g_ref, b1g_ref, w2g_ref, b2g_ref,
                 h1l_ref, h1g_ref, z4_ref):
    x = x_ref[...]
    for b, (w1_ref, b1_ref, w2_ref, b2_ref, h1_ref) in enumerate((
        (w1l_ref, b1l_ref, w2l_ref, b2l_ref, h1l_ref),
        (w1g_ref, b1g_ref, w2g_ref, b2g_ref, h1g_ref),
    )):
        h1 = lax.dot_general(x, w1_ref[...], (((1,), (1,)), ((), ())))
        h1 = jnp.maximum(h1 + b1_ref[...][None, :], 0.0)
        h1_ref[...] = h1
        z = lax.dot_general(w2_ref[...], h1, (((1,), (1,)), ((), ())))
        z = jnp.pad(z + b2_ref[...][:, None], ((0, 0), (0, NP - N)))
        z4_ref[2 * b:2 * b + 2, :] = z


_tc_mlp = pl.pallas_call(
    _tc_mlp_body,
    out_shape=(
        jax.ShapeDtypeStruct((N, H), jnp.float32),
        jax.ShapeDtypeStruct((N, H), jnp.float32),
        jax.ShapeDtypeStruct((4, NP), jnp.float32),
    ),
)


def _tc_lsm_body(h2_ref, pl_ref, pg_ref):
    v = h2_ref[...]
    for b, out_ref in ((0, pl_ref), (1, pg_ref)):
        vb = v[2 * b:2 * b + 2, :]
        m = jnp.max(vb, axis=0, keepdims=True)
        lse = m + jnp.log(jnp.sum(jnp.exp(vb - m), axis=0, keepdims=True))
        out_ref[...] = vb - lse


_tc_lsm = pl.pallas_call(
    _tc_lsm_body,
    out_shape=(
        jax.ShapeDtypeStruct((C, NP), jnp.float32),
        jax.ShapeDtypeStruct((C, NP), jnp.float32),
    ),
)


_sc_mesh = plsc.VectorSubcoreMesh(core_axis_name="c", subcore_axis_name="s")


@functools.partial(
    pl.kernel,
    out_type=jax.ShapeDtypeStruct((4 * NP,), jnp.float32),
    mesh=_sc_mesh,
    compiler_params=pltpu.CompilerParams(needs_layout_passes=False),
    scratch_types=[
        pltpu.VMEM((EPT,), jnp.int32),
        pltpu.VMEM((EPT,), jnp.int32),
        pltpu.VMEM((NP,), jnp.float32),
        pltpu.VMEM((NP,), jnp.float32),
        pltpu.VMEM((NP,), jnp.float32),
        pltpu.VMEM((NP,), jnp.float32),
        pltpu.VMEM((NTILES, NPT), jnp.float32),
        pltpu.VMEM((NTILES, NPT), jnp.float32),
        pltpu.VMEM((NPT,), jnp.float32),
        pltpu.VMEM((NPT,), jnp.float32),
        pltpu.VMEM((NPT,), jnp.float32),
        pltpu.VMEM((NPT,), jnp.float32),
        pltpu.VMEM((NPT,), jnp.float32),
        pltpu.VMEM((NPT,), jnp.float32),
        pltpu.VMEM((TREP,), jnp.float32),
        pltpu.SemaphoreType.DMA,
        pltpu.SemaphoreType.DMA,
        pltpu.SemaphoreType.DMA,
        pltpu.SemaphoreType.DMA,
        pltpu.VMEM_SHARED((2, NTILES, NP), jnp.float32),
        pltpu.VMEM_SHARED((2, NP), jnp.float32),
    ],
)
def _sc_prop(src_hbm, dst_hbm, z_all, temp_rep, out,
             src_v, dst_v, y0_v, y1_v, s0_v, s1_v, part_v, part2_v,
             ysl0_v, ysl1_v, hid0_v, hid1_v, dinv_v, red_v, tsp_v,
             sem0, sem1, sem2, sem3, sh_p, sh_y):
    c = lax.axis_index("c")
    sid = lax.axis_index("s")
    zeros16 = jnp.zeros((16,), jnp.float32)
    ones16 = jnp.ones((16,), jnp.float32)
    n0 = sid * NPT

    pltpu.sync_copy(src_hbm.at[pl.ds(sid * EPT, EPT)], src_v)
    pltpu.sync_copy(dst_hbm.at[pl.ds(sid * EPT, EPT)], dst_v)
    pltpu.sync_copy(temp_rep.at[pl.ds(c * TREP, TREP)], tsp_v)

    def zero_buf(buf):
        @plsc.parallel_loop(0, NP // 16, 1, unroll=8)
        def zero_body(i):
            buf[pl.ds(i * 16, 16)] = zeros16

    def zero_s(_=None):
        zero_buf(s0_v)
        zero_buf(s1_v)

    def sum_slab(slab, dst_buf):
        @plsc.parallel_loop(0, NPT // 16, 1, unroll=4)
        def red_body(i):
            acc = slab[0, pl.ds(i * 16, 16)]
            for t in range(1, NTILES):
                acc = acc + slab[t, pl.ds(i * 16, 16)]
            dst_buf[pl.ds(i * 16, 16)] = acc

    def reduce_partials(ch, dst_buf):
        pltpu.sync_copy(sh_p.at[ch, pl.ds(0, NTILES), pl.ds(n0, NPT)],
                        part_v)
        sum_slab(part_v, dst_buf)

    zero_s()

    @plsc.parallel_loop(0, EPT // 16, 1, unroll=EDGE_UNROLL)
    def deg_body(i):
        d16 = dst_v[pl.ds(i * 16, 16)]
        plsc.addupdate_scatter(s0_v, [d16], ones16)
    pltpu.sync_copy(s0_v, sh_p.at[0, sid])
    plsc.subcore_barrier()

    reduce_partials(0, red_v)

    def dinv_body(i, _):
        d = red_v[pl.ds(i * 16, 16)] + 1.0
        dinv_v[pl.ds(i * 16, 16)] = _rsqrt_newton(d)
        return 0

    lax.fori_loop(0, NPT // 16, dinv_body, 0)

    @plsc.parallel_loop(0, NP // 16, 1, unroll=8)
    def zero_s0_body(i):
        s0_v[pl.ds(i * 16, 16)] = zeros16

    coef0 = tsp_v[pl.ds(0, 16)]
    for ch, (ysl, hid) in enumerate(((ysl0_v, hid0_v), (ysl1_v, hid1_v))):
        pltpu.sync_copy(z_all.at[pl.ds((2 * c + ch) * NP + n0, NPT)], ysl)

        def init_body(i, _, ysl=ysl, hid=hid):
            x0 = ysl[pl.ds(i * 16, 16)]
            hid[pl.ds(i * 16, 16)] = coef0 * x0
            ysl[pl.ds(i * 16, 16)] = dinv_v[pl.ds(i * 16, 16)] * x0
            return 0

        lax.fori_loop(0, NPT // 16, init_body, 0)
        pltpu.sync_copy(ysl, sh_y.at[ch, pl.ds(n0, NPT)])
    plsc.subcore_barrier()
    pltpu.sync_copy(sh_y.at[0], y0_v)
    pltpu.sync_copy(sh_y.at[1], y1_v)

    def step_body(k, _):
        @plsc.parallel_loop(0, EPT // 16, 1, unroll=EDGE_UNROLL)
        def edge_body(i):
            s16 = src_v[pl.ds(i * 16, 16)]
            d16 = dst_v[pl.ds(i * 16, 16)]
            g0 = plsc.load_gather(y0_v, [s16])
            g1 = plsc.load_gather(y1_v, [s16])
            plsc.addupdate_scatter(s0_v, [d16], g0)
            plsc.addupdate_scatter(s1_v, [d16], g1)

        cp0 = pltpu.async_copy(s0_v, sh_p.at[0, sid], sem0)
        cp1 = pltpu.async_copy(s1_v, sh_p.at[1, sid], sem1)
        cp0.wait()
        zero_buf(s0_v)
        cp1.wait()
        zero_buf(s1_v)
        plsc.subcore_barrier()

        coef = tsp_v[pl.ds(16 * (k + 1), 16)]
        sl0 = pltpu.async_copy(sh_p.at[0, pl.ds(0, NTILES), pl.ds(n0, NPT)],
                               part_v, sem0)
        sl1 = pltpu.async_copy(sh_p.at[1, pl.ds(0, NTILES), pl.ds(n0, NPT)],
                               part2_v, sem1)
        pubs = []
        for ch, (ysl, hid, slab, sl) in enumerate((
                (ysl0_v, hid0_v, part_v, sl0),
                (ysl1_v, hid1_v, part2_v, sl1))):
            sl.wait()
            sum_slab(slab, red_v)

            @plsc.parallel_loop(0, NPT // 16, 1, unroll=4)
            def upd_body(i, ysl=ysl, hid=hid, coef=coef):
                dv = dinv_v[pl.ds(i * 16, 16)]
                t = red_v[pl.ds(i * 16, 16)] + ysl[pl.ds(i * 16, 16)]
                xp = dv * t
                hid[pl.ds(i * 16, 16)] = hid[pl.ds(i * 16, 16)] + coef * xp
                ysl[pl.ds(i * 16, 16)] = dv * xp

            pubs.append(pltpu.async_copy(
                ysl, sh_y.at[ch, pl.ds(n0, NPT)], sem2 if ch == 0 else sem3))

        for p in pubs:
            p.wait()
        plsc.subcore_barrier()
        rd0 = pltpu.async_copy(sh_y.at[0], y0_v, sem0)
        rd1 = pltpu.async_copy(sh_y.at[1], y1_v, sem1)
        rd0.wait()
        rd1.wait()
        return 0

    lax.fori_loop(0, K_G, step_body, 0)

    pltpu.sync_copy(hid0_v, out.at[pl.ds(2 * c * NP + n0, NPT)])
    pltpu.sync_copy(hid1_v, out.at[pl.ds((2 * c + 1) * NP + n0, NPT)])


def kernel(x, edge_index, W1_l, b1_l, W2_l, b2_l, temp_l,
           W1_g, b1_g, W2_g, b2_g, temp_g):
    h1_l, h1_g, z4 = _tc_mlp(x, W1_l, b1_l, W2_l, b2_l,
                             W1_g, b1_g, W2_g, b2_g)

    z_all = z4.reshape(-1)
    tl = jnp.pad(temp_l, (0, K_G + 1 - temp_l.shape[0]))
    tg = jnp.pad(temp_g, (0, K_G + 1 - temp_g.shape[0]))
    temp_rep = jnp.concatenate([
        jnp.repeat(tl, 16), jnp.repeat(tg, 16)])

    hid = _sc_prop(edge_index[0], edge_index[1], z_all, temp_rep)
    hid2d = hid.reshape(4, NP)

    plT, pgT = _tc_lsm(hid2d)

    h2_l = hid2d[0:2, :N].T
    h2_g = hid2d[2:4, :N].T
    p_l = plT[:, :N].T
    p_g = pgT[:, :N].T
    return (h1_l, h2_l, p_l, h1_g, h2_g, p_g)
